# baseline (speedup 1.0000x reference)
import functools

import jax
import jax.numpy as jnp
from jax.experimental import pallas as pl
from jax.experimental.pallas import tpu as pltpu

_BM = 256


def _bmm_kernel(a_ref, h_ref, o_ref):
    o_ref[0] = jnp.dot(a_ref[0], h_ref[0], preferred_element_type=jnp.float32)


@jax.jit
def kernel(h_text, alignment):
    B, Lm, Lt = alignment.shape
    Ht = h_text.shape[2]
    grid = (B, Lm // _BM)
    return pl.pallas_call(
        _bmm_kernel,
        grid=grid,
        in_specs=[
            pl.BlockSpec((1, _BM, Lt), lambda b, i: (b, i, 0)),
            pl.BlockSpec((1, Lt, Ht), lambda b, i: (b, 0, 0)),
        ],
        out_specs=pl.BlockSpec((1, _BM, Ht), lambda b, i: (b, i, 0)),
        out_shape=jax.ShapeDtypeStruct((B, Lm, Ht), jnp.float32),
        compiler_params=pltpu.CompilerParams(
            dimension_semantics=("parallel", "parallel"),
        ),
    )(alignment, h_text)


# speedup vs baseline: 2.1341x; 2.1341x over previous
import functools

import jax
import jax.numpy as jnp
from jax.experimental import pallas as pl
from jax.experimental.pallas import tpu as pltpu

_BM = 1024


def _bmm_kernel(a_ref, h_ref, o_ref):
    o_ref[0] = jnp.dot(a_ref[0], h_ref[0], preferred_element_type=jnp.float32)


@jax.jit
def kernel(h_text, alignment):
    B, Lm, Lt = alignment.shape
    Ht = h_text.shape[2]
    grid = (B, Lm // _BM)
    return pl.pallas_call(
        _bmm_kernel,
        grid=grid,
        in_specs=[
            pl.BlockSpec((1, _BM, Lt), lambda b, i: (b, i, 0)),
            pl.BlockSpec((1, Lt, Ht), lambda b, i: (b, 0, 0)),
        ],
        out_specs=pl.BlockSpec((1, _BM, Ht), lambda b, i: (b, i, 0)),
        out_shape=jax.ShapeDtypeStruct((B, Lm, Ht), jnp.float32),
        compiler_params=pltpu.CompilerParams(
            dimension_semantics=("parallel", "parallel"),
        ),
    )(alignment, h_text)


# speedup vs baseline: 2.5737x; 1.2060x over previous
import functools

import jax
import jax.numpy as jnp
from jax.experimental import pallas as pl
from jax.experimental.pallas import tpu as pltpu

_BM = 2048


def _bmm_kernel(a_ref, h_ref, o_ref):
    o_ref[0] = jnp.dot(a_ref[0], h_ref[0], preferred_element_type=jnp.float32)


@jax.jit
def kernel(h_text, alignment):
    B, Lm, Lt = alignment.shape
    Ht = h_text.shape[2]
    grid = (B, Lm // _BM)
    return pl.pallas_call(
        _bmm_kernel,
        grid=grid,
        in_specs=[
            pl.BlockSpec((1, _BM, Lt), lambda b, i: (b, i, 0)),
            pl.BlockSpec((1, Lt, Ht), lambda b, i: (b, 0, 0)),
        ],
        out_specs=pl.BlockSpec((1, _BM, Ht), lambda b, i: (b, i, 0)),
        out_shape=jax.ShapeDtypeStruct((B, Lm, Ht), jnp.float32),
        compiler_params=pltpu.CompilerParams(
            dimension_semantics=("parallel", "parallel"),
        ),
    )(alignment, h_text)


# speedup vs baseline: 2.6898x; 1.0451x over previous
import jax
import jax.numpy as jnp
from jax.experimental import pallas as pl
from jax.experimental.pallas import tpu as pltpu

_BB = 2


def _bmm_kernel(a_ref, h_ref, o_ref):
    for j in range(_BB):
        o_ref[j] = jnp.dot(a_ref[j], h_ref[j], preferred_element_type=jnp.float32)


@jax.jit
def kernel(h_text, alignment):
    B, Lm, Lt = alignment.shape
    Ht = h_text.shape[2]
    grid = (B // _BB,)
    return pl.pallas_call(
        _bmm_kernel,
        grid=grid,
        in_specs=[
            pl.BlockSpec((_BB, Lm, Lt), lambda b: (b, 0, 0)),
            pl.BlockSpec((_BB, Lt, Ht), lambda b: (b, 0, 0)),
        ],
        out_specs=pl.BlockSpec((_BB, Lm, Ht), lambda b: (b, 0, 0)),
        out_shape=jax.ShapeDtypeStruct((B, Lm, Ht), jnp.float32),
        compiler_params=pltpu.CompilerParams(
            dimension_semantics=("arbitrary",),
        ),
    )(alignment, h_text)
